# Initial kernel scaffold; baseline (speedup 1.0000x reference)
import functools

import numpy as np
import jax
import jax.numpy as jnp
from jax import lax
from jax.experimental import pallas as pl
from jax.experimental.pallas import tpu as pltpu
from jax.experimental.pallas import tpu_sc as plsc

Nx = 512
Ny = 512
Nu = 768
NTHETA = 90
NC = 2
NS = 16
L = 16
VPC = NTHETA // NC
RPT = Ny // NS
XB = Nx // L
ACC_N = VPC * Nu
SLICE = ACC_N // NS


def _make_tables():
    th = np.arange(NTHETA, dtype=np.float32) * np.float32(np.pi / NTHETA)
    cos_t = np.cos(th)
    sin_t = np.sin(th)
    ac, asn = np.abs(cos_t), np.abs(sin_t)
    h = np.minimum(1.0 / np.maximum(ac, 1e-12),
                   1.0 / np.maximum(asn, 1e-12)).astype(np.float32)
    b1 = np.abs(asn - ac)
    b2 = asn + ac
    r1 = (b2 - b1) * np.float32(0.5)
    r2 = (b2 + b1) * np.float32(0.5)
    r3 = b2
    bigA = h / (2.0 * np.maximum(r1, np.float32(1e-6)))
    atot = h * (b1 + b2) * np.float32(0.5)
    c0 = (-(Nx - 1) / 2.0) * cos_t - b2 * np.float32(0.5) \
        + np.float32((Nu - 1) / 2.0 + 256.0)
    par = np.stack([cos_t, sin_t, c0, r1, r2, r3, bigA, h, atot], axis=1)
    par = np.repeat(par.astype(np.float32)[:, :, None], L, axis=2)
    xtab = np.arange(Nx, dtype=np.float32).reshape(XB, L)
    ytab = np.repeat(((np.arange(Ny, dtype=np.float32) - (Ny - 1) / 2.0)
                      )[:, None], L, axis=1)
    return par, xtab, ytab


_PAR, _XTAB, _YTAB = _make_tables()

_mesh = plsc.VectorSubcoreMesh(core_axis_name="c", subcore_axis_name="s")


@functools.partial(
    pl.kernel,
    out_type=jax.ShapeDtypeStruct((NTHETA * Nu,), jnp.float32),
    mesh=_mesh,
    scratch_types=[
        pltpu.VMEM((RPT, Nx), jnp.float32),
        pltpu.VMEM((VPC, 9, L), jnp.float32),
        pltpu.VMEM((XB, L), jnp.float32),
        pltpu.VMEM((RPT, L), jnp.float32),
        pltpu.VMEM((NS * Nu,), jnp.float32),
        pltpu.VMEM((ACC_N,), jnp.float32),
        pltpu.VMEM((SLICE,), jnp.float32),
        pltpu.VMEM((SLICE,), jnp.float32),
        pltpu.VMEM_SHARED((NS, ACC_N), jnp.float32),
    ],
)
def _ct_project_sc(img_h, par_h, xtab_h, ytab_h, out_h,
                   img_v, par_v, xtab_v, ytab_v, acc16, accv,
                   red_a, red_t, slots):
    c = lax.axis_index("c")
    s = lax.axis_index("s")
    pltpu.sync_copy(img_h.at[pl.ds(s * RPT, RPT)], img_v)
    pltpu.sync_copy(par_h.at[pl.ds(c * VPC, VPC)], par_v)
    pltpu.sync_copy(xtab_h, xtab_v)
    pltpu.sync_copy(ytab_h.at[pl.ds(s * RPT, RPT)], ytab_v)

    zero = jnp.zeros((L,), jnp.float32)
    lane_off = jnp.arange(L, dtype=jnp.int32) * Nu - 256
    one_i = jnp.full((L,), 1, jnp.int32)

    @pl.loop(0, VPC)
    def _view(v):
        alpha = par_v[v, 0, :]
        beta = par_v[v, 1, :]
        c0 = par_v[v, 2, :]
        r1 = par_v[v, 3, :]
        r2 = par_v[v, 4, :]
        r3 = par_v[v, 5, :]
        bigA = par_v[v, 6, :]
        h = par_v[v, 7, :]
        atot = par_v[v, 8, :]

        @pl.loop(0, NS * Nu // L, unroll=8)
        def _zero(i):
            acc16[pl.ds(i * L, L)] = zero

        @pl.loop(0, RPT)
        def _row(r):
            cr = c0 + beta * ytab_v[r, :]

            @pl.loop(0, XB, unroll=4)
            def _xblk(xb):
                u1i = alpha * xtab_v[xb, :] + cr
                bi = u1i.astype(jnp.int32)
                f = u1i - bi.astype(jnp.float32)
                t1 = 0.5 - f
                t2 = 1.5 - f

                def F(t):
                    c1 = jnp.minimum(jnp.maximum(t, 0.0), r1)
                    c2 = jnp.minimum(jnp.maximum(t, r1), r2) - r1
                    c3 = jnp.minimum(jnp.maximum(t, r2), r3) - r2
                    return bigA * ((c1 - c3) * (c1 + c3)) + h * (c2 + c3)

                F1 = F(t1)
                F2 = F(t2)
                g = img_v[r, pl.ds(xb * L, L)]
                w0 = F1 * g
                w1 = (F2 - F1) * g
                w2 = (atot - F2) * g
                i0 = bi + lane_off
                i1 = i0 + one_i
                i2 = i1 + one_i
                plsc.addupdate_scatter(acc16, [i0], w0)
                plsc.addupdate_scatter(acc16, [i1], w1)
                plsc.addupdate_scatter(acc16, [i2], w2)

        @pl.loop(0, Nu // L)
        def _reduce(j):
            t = acc16[pl.ds(j * L, L)]
            for lane in range(1, NS):
                t = t + acc16[pl.ds(lane * Nu + j * L, L)]
            accv[pl.ds(v * Nu + j * L, L)] = t

    pltpu.sync_copy(accv, slots.at[s])
    plsc.subcore_barrier()
    pltpu.sync_copy(slots.at[0, pl.ds(s * SLICE, SLICE)], red_a)

    @pl.loop(1, NS)
    def _slot(k):
        pltpu.sync_copy(slots.at[k, pl.ds(s * SLICE, SLICE)], red_t)

        @pl.loop(0, SLICE // L, unroll=8)
        def _add(i):
            red_a[pl.ds(i * L, L)] = red_a[pl.ds(i * L, L)] \
                + red_t[pl.ds(i * L, L)]

    pltpu.sync_copy(red_a, out_h.at[pl.ds(c * ACC_N + s * SLICE, SLICE)])


def kernel(img):
    out = _ct_project_sc(img, jnp.asarray(_PAR), jnp.asarray(_XTAB),
                         jnp.asarray(_YTAB))
    return out.reshape(NTHETA, Nu)


# speedup vs baseline: 89.8414x; 89.8414x over previous
import functools

import numpy as np
import jax
import jax.numpy as jnp
from jax import lax
from jax.experimental import pallas as pl
from jax.experimental.pallas import tpu as pltpu
from jax.experimental.pallas import tpu_sc as plsc

Nx = 512
Ny = 512
Nu = 768
NTHETA = 90
NC = 2
NS = 16
L = 16
VPC = NTHETA // NC
RPT = Ny // NS
XB = Nx // L
ACC_N = VPC * Nu
SLICE = ACC_N // NS


def _make_tables():
    th = np.arange(NTHETA, dtype=np.float32) * np.float32(np.pi / NTHETA)
    cos_t = np.cos(th)
    sin_t = np.sin(th)
    ac, asn = np.abs(cos_t), np.abs(sin_t)
    h = np.minimum(1.0 / np.maximum(ac, 1e-12),
                   1.0 / np.maximum(asn, 1e-12)).astype(np.float32)
    b1 = np.abs(asn - ac)
    b2 = asn + ac
    r1 = (b2 - b1) * np.float32(0.5)
    r2 = (b2 + b1) * np.float32(0.5)
    r3 = b2
    bigA = h / (2.0 * np.maximum(r1, np.float32(1e-6)))
    atot = h * (b1 + b2) * np.float32(0.5)
    c0 = (-(Nx - 1) / 2.0) * cos_t - b2 * np.float32(0.5) \
        + np.float32((Nu - 1) / 2.0 + 256.0)
    par = np.stack([cos_t, sin_t, c0, r1, r2, r3, bigA, h, atot], axis=1)
    par = np.repeat(par.astype(np.float32)[:, :, None], L, axis=2)
    xtab = np.arange(Nx, dtype=np.float32).reshape(XB, L)
    ytab = np.repeat(((np.arange(Ny, dtype=np.float32) - (Ny - 1) / 2.0)
                      )[:, None], L, axis=1)
    return par, xtab, ytab


_PAR, _XTAB, _YTAB = _make_tables()

_mesh = plsc.VectorSubcoreMesh(core_axis_name="c", subcore_axis_name="s")


@functools.partial(
    pl.kernel,
    out_type=jax.ShapeDtypeStruct((NTHETA * Nu,), jnp.float32),
    mesh=_mesh,
    scratch_types=[
        pltpu.VMEM((RPT, Nx), jnp.float32),
        pltpu.VMEM((VPC, 9, L), jnp.float32),
        pltpu.VMEM((XB, L), jnp.float32),
        pltpu.VMEM((RPT, L), jnp.float32),
        pltpu.VMEM((NS * Nu,), jnp.float32),
        pltpu.VMEM((ACC_N,), jnp.float32),
        pltpu.VMEM((SLICE,), jnp.float32),
        pltpu.VMEM((SLICE,), jnp.float32),
        pltpu.VMEM_SHARED((NS, ACC_N), jnp.float32),
    ],
    compiler_params=pltpu.CompilerParams(use_tc_tiling_on_sc=False,
                                         needs_layout_passes=False),
)
def _ct_project_sc(img_h, par_h, xtab_h, ytab_h, out_h,
                   img_v, par_v, xtab_v, ytab_v, acc16, accv,
                   red_a, red_t, slots):
    c = lax.axis_index("c")
    s = lax.axis_index("s")
    pltpu.sync_copy(img_h.at[pl.ds(s * RPT, RPT)], img_v)
    pltpu.sync_copy(par_h.at[pl.ds(c * VPC, VPC)], par_v)
    pltpu.sync_copy(xtab_h, xtab_v)
    pltpu.sync_copy(ytab_h.at[pl.ds(s * RPT, RPT)], ytab_v)

    zero = jnp.zeros((L,), jnp.float32)
    lane_off = jnp.arange(L, dtype=jnp.int32) * Nu - 256
    one_i = jnp.full((L,), 1, jnp.int32)

    @pl.loop(0, VPC)
    def _view(v):
        alpha = par_v[v, 0, :]
        beta = par_v[v, 1, :]
        c0 = par_v[v, 2, :]
        r1 = par_v[v, 3, :]
        r2 = par_v[v, 4, :]
        r3 = par_v[v, 5, :]
        bigA = par_v[v, 6, :]
        h = par_v[v, 7, :]
        atot = par_v[v, 8, :]

        @pl.loop(0, NS * Nu // L, unroll=8)
        def _zero(i):
            acc16[pl.ds(i * L, L)] = zero

        @pl.loop(0, RPT)
        def _row(r):
            cr = c0 + beta * ytab_v[r, :]

            @pl.loop(0, XB, unroll=4)
            def _xblk(xb):
                u1i = alpha * xtab_v[xb, :] + cr
                bi = u1i.astype(jnp.int32)
                f = u1i - bi.astype(jnp.float32)
                t1 = 0.5 - f
                t2 = 1.5 - f

                def F(t):
                    c1 = jnp.minimum(jnp.maximum(t, 0.0), r1)
                    c2 = jnp.minimum(jnp.maximum(t, r1), r2) - r1
                    c3 = jnp.minimum(jnp.maximum(t, r2), r3) - r2
                    return bigA * ((c1 - c3) * (c1 + c3)) + h * (c2 + c3)

                F1 = F(t1)
                F2 = F(t2)
                g = img_v[r, pl.ds(xb * L, L)]
                w0 = F1 * g
                w1 = (F2 - F1) * g
                w2 = (atot - F2) * g
                i0 = bi + lane_off
                i1 = i0 + one_i
                i2 = i1 + one_i
                plsc.addupdate_scatter(acc16, [i0], w0)
                plsc.addupdate_scatter(acc16, [i1], w1)
                plsc.addupdate_scatter(acc16, [i2], w2)

        @pl.loop(0, Nu // L)
        def _reduce(j):
            t = acc16[pl.ds(j * L, L)]
            for lane in range(1, NS):
                t = t + acc16[pl.ds(lane * Nu + j * L, L)]
            accv[pl.ds(v * Nu + j * L, L)] = t

    pltpu.sync_copy(accv, slots.at[s])
    plsc.subcore_barrier()
    pltpu.sync_copy(slots.at[0, pl.ds(s * SLICE, SLICE)], red_a)

    @pl.loop(1, NS)
    def _slot(k):
        pltpu.sync_copy(slots.at[k, pl.ds(s * SLICE, SLICE)], red_t)

        @pl.loop(0, SLICE // L, unroll=8)
        def _add(i):
            red_a[pl.ds(i * L, L)] = red_a[pl.ds(i * L, L)] \
                + red_t[pl.ds(i * L, L)]

    pltpu.sync_copy(red_a, out_h.at[pl.ds(c * ACC_N + s * SLICE, SLICE)])


def kernel(img):
    out = _ct_project_sc(img, jnp.asarray(_PAR), jnp.asarray(_XTAB),
                         jnp.asarray(_YTAB))
    return out.reshape(NTHETA, Nu)


# speedup vs baseline: 90.4053x; 1.0063x over previous
import functools

import numpy as np
import jax
import jax.numpy as jnp
from jax import lax
from jax.experimental import pallas as pl
from jax.experimental.pallas import tpu as pltpu
from jax.experimental.pallas import tpu_sc as plsc

Nx = 512
Ny = 512
Nu = 768
NTHETA = 90
NC = 2
NS = 16
L = 16
VPC = NTHETA // NC
RPT = Ny // NS
XB = Nx // L
ACC_N = VPC * Nu
SLICE = ACC_N // NS


def _make_tables():
    th = np.arange(NTHETA, dtype=np.float32) * np.float32(np.pi / NTHETA)
    cos_t = np.cos(th)
    sin_t = np.sin(th)
    ac, asn = np.abs(cos_t), np.abs(sin_t)
    h = np.minimum(1.0 / np.maximum(ac, 1e-12),
                   1.0 / np.maximum(asn, 1e-12)).astype(np.float32)
    b1 = np.abs(asn - ac)
    b2 = asn + ac
    r1 = (b2 - b1) * np.float32(0.5)
    r2 = (b2 + b1) * np.float32(0.5)
    r3 = b2
    bigA = h / (2.0 * np.maximum(r1, np.float32(1e-6)))
    atot = h * (b1 + b2) * np.float32(0.5)
    c0 = (-(Nx - 1) / 2.0) * cos_t - b2 * np.float32(0.5) \
        + np.float32((Nu - 1) / 2.0 + 256.0)
    par = np.stack([cos_t, sin_t, c0, r1, r2, r3, bigA, h, atot], axis=1)
    par = np.repeat(par.astype(np.float32)[:, :, None], L, axis=2)
    xtab = np.arange(Nx, dtype=np.float32).reshape(XB, L)
    ytab = np.repeat(((np.arange(Ny, dtype=np.float32) - (Ny - 1) / 2.0)
                      )[:, None], L, axis=1)
    return par, xtab, ytab


_PAR, _XTAB, _YTAB = _make_tables()

_mesh = plsc.VectorSubcoreMesh(core_axis_name="c", subcore_axis_name="s")


@functools.partial(
    pl.kernel,
    out_type=jax.ShapeDtypeStruct((NTHETA * Nu,), jnp.float32),
    mesh=_mesh,
    scratch_types=[
        pltpu.VMEM((RPT, Nx), jnp.float32),
        pltpu.VMEM((VPC, 9, L), jnp.float32),
        pltpu.VMEM((XB, L), jnp.float32),
        pltpu.VMEM((RPT, L), jnp.float32),
        pltpu.VMEM((NS * Nu,), jnp.float32),
        pltpu.VMEM((ACC_N,), jnp.float32),
        pltpu.VMEM((SLICE,), jnp.float32),
        pltpu.VMEM((SLICE,), jnp.float32),
        pltpu.VMEM_SHARED((NS, ACC_N), jnp.float32),
    ],
    compiler_params=pltpu.CompilerParams(use_tc_tiling_on_sc=False,
                                         needs_layout_passes=False),
)
def _ct_project_sc(img_h, par_h, xtab_h, ytab_h, out_h,
                   img_v, par_v, xtab_v, ytab_v, acc16, accv,
                   red_a, red_t, slots):
    c = lax.axis_index("c")
    s = lax.axis_index("s")
    pltpu.sync_copy(img_h.at[pl.ds(s * RPT, RPT)], img_v)
    pltpu.sync_copy(par_h.at[pl.ds(c * VPC, VPC)], par_v)
    pltpu.sync_copy(xtab_h, xtab_v)
    pltpu.sync_copy(ytab_h.at[pl.ds(s * RPT, RPT)], ytab_v)

    zero = jnp.zeros((L,), jnp.float32)
    lane_off = jnp.arange(L, dtype=jnp.int32) * Nu - 256
    one_i = jnp.full((L,), 1, jnp.int32)

    @pl.loop(0, NS * Nu // L, unroll=8)
    def _zero(i):
        acc16[pl.ds(i * L, L)] = zero

    @pl.loop(0, VPC)
    def _view(v):
        alpha = par_v[v, 0, :]
        beta = par_v[v, 1, :]
        c0 = par_v[v, 2, :]
        r1 = par_v[v, 3, :]
        r2 = par_v[v, 4, :]
        r3 = par_v[v, 5, :]
        bigA = par_v[v, 6, :]
        h = par_v[v, 7, :]
        atot = par_v[v, 8, :]

        @pl.loop(0, RPT)
        def _row(r):
            cr = c0 + beta * ytab_v[r, :]

            @pl.loop(0, XB, unroll=8)
            def _xblk(xb):
                u1i = alpha * xtab_v[xb, :] + cr
                bi = u1i.astype(jnp.int32)
                f = u1i - bi.astype(jnp.float32)
                t1 = 0.5 - f
                t2 = 1.5 - f

                def F(t):
                    c1 = jnp.minimum(jnp.maximum(t, 0.0), r1)
                    c2 = jnp.minimum(jnp.maximum(t, r1), r2) - r1
                    c3 = jnp.minimum(jnp.maximum(t, r2), r3) - r2
                    return bigA * ((c1 - c3) * (c1 + c3)) + h * (c2 + c3)

                F1 = F(t1)
                F2 = F(t2)
                g = img_v[r, pl.ds(xb * L, L)]
                w0 = F1 * g
                w1 = (F2 - F1) * g
                w2 = (atot - F2) * g
                i0 = bi + lane_off
                i1 = i0 + one_i
                i2 = i1 + one_i
                plsc.addupdate_scatter(acc16, [i0], w0)
                plsc.addupdate_scatter(acc16, [i1], w1)
                plsc.addupdate_scatter(acc16, [i2], w2)

        @pl.loop(0, Nu // L, unroll=2)
        def _reduce(j):
            t = acc16[pl.ds(j * L, L)]
            acc16[pl.ds(j * L, L)] = zero
            for lane in range(1, NS):
                t = t + acc16[pl.ds(lane * Nu + j * L, L)]
                acc16[pl.ds(lane * Nu + j * L, L)] = zero
            accv[pl.ds(v * Nu + j * L, L)] = t

    pltpu.sync_copy(accv, slots.at[s])
    plsc.subcore_barrier()
    pltpu.sync_copy(slots.at[0, pl.ds(s * SLICE, SLICE)], red_a)

    @pl.loop(1, NS)
    def _slot(k):
        pltpu.sync_copy(slots.at[k, pl.ds(s * SLICE, SLICE)], red_t)

        @pl.loop(0, SLICE // L, unroll=8)
        def _add(i):
            red_a[pl.ds(i * L, L)] = red_a[pl.ds(i * L, L)] \
                + red_t[pl.ds(i * L, L)]

    pltpu.sync_copy(red_a, out_h.at[pl.ds(c * ACC_N + s * SLICE, SLICE)])


def kernel(img):
    out = _ct_project_sc(img, jnp.asarray(_PAR), jnp.asarray(_XTAB),
                         jnp.asarray(_YTAB))
    return out.reshape(NTHETA, Nu)


# speedup vs baseline: 205.0286x; 2.2679x over previous
import functools

import numpy as np
import jax
import jax.numpy as jnp
from jax import lax
from jax.experimental import pallas as pl
from jax.experimental.pallas import tpu as pltpu
from jax.experimental.pallas import tpu_sc as plsc

Nx = 512
Ny = 512
Nu = 768
NTHETA = 90
NC = 2
NS = 16
L = 16
VPC = NTHETA // NC
RPT = Ny // NS
XB = Nx // L
ACC_N = VPC * Nu
SLICE = ACC_N // NS


def _make_tables():
    th = np.arange(NTHETA, dtype=np.float32) * np.float32(np.pi / NTHETA)
    cos_t = np.cos(th)
    sin_t = np.sin(th)
    ac, asn = np.abs(cos_t), np.abs(sin_t)
    h = np.minimum(1.0 / np.maximum(ac, 1e-12),
                   1.0 / np.maximum(asn, 1e-12)).astype(np.float32)
    b1 = np.abs(asn - ac)
    b2 = asn + ac
    r1 = (b2 - b1) * np.float32(0.5)
    r2 = (b2 + b1) * np.float32(0.5)
    r3 = b2
    bigA = h / (2.0 * np.maximum(r1, np.float32(1e-6)))
    atot = h * (b1 + b2) * np.float32(0.5)
    c0 = (-(Nx - 1) / 2.0) * cos_t - b2 * np.float32(0.5) \
        + np.float32((Nu - 1) / 2.0 + 256.0)
    par = np.stack([cos_t, sin_t, c0, r1, r2, r3, bigA, h, atot], axis=1)
    par = np.repeat(par.astype(np.float32)[:, :, None], L, axis=2)
    xtab = np.arange(Nx, dtype=np.float32).reshape(XB, L)
    ytab = np.repeat(((np.arange(Ny, dtype=np.float32) - (Ny - 1) / 2.0)
                      )[:, None], L, axis=1)
    return par, xtab, ytab


_PAR, _XTAB, _YTAB = _make_tables()

_mesh = plsc.VectorSubcoreMesh(core_axis_name="c", subcore_axis_name="s")


@functools.partial(
    pl.kernel,
    out_type=jax.ShapeDtypeStruct((NTHETA * Nu,), jnp.float32),
    mesh=_mesh,
    scratch_types=[
        pltpu.VMEM((RPT, Nx), jnp.float32),
        pltpu.VMEM((VPC, 9, L), jnp.float32),
        pltpu.VMEM((XB, L), jnp.float32),
        pltpu.VMEM((RPT, L), jnp.float32),
        pltpu.VMEM((NS * Nu,), jnp.float32),
        pltpu.VMEM((ACC_N,), jnp.float32),
        pltpu.VMEM((SLICE,), jnp.float32),
        pltpu.VMEM((SLICE,), jnp.float32),
        pltpu.VMEM_SHARED((NS, ACC_N), jnp.float32),
    ],
    compiler_params=pltpu.CompilerParams(use_tc_tiling_on_sc=False,
                                         needs_layout_passes=False),
)
def _ct_project_sc(img_h, par_h, xtab_h, ytab_h, out_h,
                   img_v, par_v, xtab_v, ytab_v, acc16, accv,
                   red_a, red_t, slots):
    c = lax.axis_index("c")
    s = lax.axis_index("s")
    pltpu.sync_copy(img_h.at[pl.ds(s * RPT, RPT)], img_v)
    pltpu.sync_copy(par_h.at[pl.ds(c * VPC, VPC)], par_v)
    pltpu.sync_copy(xtab_h, xtab_v)
    pltpu.sync_copy(ytab_h.at[pl.ds(s * RPT, RPT)], ytab_v)

    zero = jnp.zeros((L,), jnp.float32)
    lane_off0 = jnp.arange(L, dtype=jnp.int32) * Nu - 256
    lane_off1 = lane_off0 + 1
    lane_off2 = lane_off0 + 2
    lane_f = jnp.arange(L, dtype=jnp.int32).astype(jnp.float32)

    @pl.loop(0, NS * Nu // L, unroll=8)
    def _zero(i):
        acc16[pl.ds(i * L, L)] = zero

    @pl.loop(0, VPC)
    def _view(v):
        alpha = par_v[v, 0, :]
        beta = par_v[v, 1, :]
        c0 = par_v[v, 2, :]
        r1 = par_v[v, 3, :]
        r2 = par_v[v, 4, :]
        r3 = par_v[v, 5, :]
        bigA = par_v[v, 6, :]
        h = par_v[v, 7, :]
        atot = par_v[v, 8, :]
        step16 = alpha * np.float32(L)

        @pl.loop(0, RPT)
        def _row(r):
            cr = c0 + beta * ytab_v[r, :]
            u1i0 = alpha * lane_f + cr

            @pl.loop(0, XB, init_carry=u1i0, unroll=8)
            def _xblk(xb, u1i):
                bi = u1i.astype(jnp.int32)
                bf = bi.astype(jnp.float32)
                t1 = (bf - u1i) + np.float32(0.5)
                t2 = t1 + np.float32(1.0)
                c1a = jnp.maximum(jnp.minimum(t1, r1), np.float32(0.0))
                c2a = jnp.maximum(t1, r1) - r1
                F1 = bigA * (c1a * c1a) + h * c2a
                c1b = jnp.minimum(t2, r1)
                c2b = jnp.maximum(jnp.minimum(t2, r2), r1) - r1
                c3b = jnp.maximum(jnp.minimum(t2, r3), r2) - r2
                F2 = bigA * ((c1b - c3b) * (c1b + c3b)) + h * (c2b + c3b)
                g = img_v[r, pl.ds(xb * L, L)]
                w0 = F1 * g
                w1 = (F2 - F1) * g
                w2 = (atot - F2) * g
                plsc.addupdate_scatter(acc16, [bi + lane_off0], w0)
                plsc.addupdate_scatter(acc16, [bi + lane_off1], w1)
                plsc.addupdate_scatter(acc16, [bi + lane_off2], w2)
                return u1i + step16

        @pl.loop(0, Nu // L, unroll=2)
        def _reduce(j):
            t = acc16[pl.ds(j * L, L)]
            acc16[pl.ds(j * L, L)] = zero
            for lane in range(1, NS):
                t = t + acc16[pl.ds(lane * Nu + j * L, L)]
                acc16[pl.ds(lane * Nu + j * L, L)] = zero
            accv[pl.ds(v * Nu + j * L, L)] = t

    pltpu.sync_copy(accv, slots.at[s])
    plsc.subcore_barrier()
    pltpu.sync_copy(slots.at[0, pl.ds(s * SLICE, SLICE)], red_a)

    @pl.loop(1, NS)
    def _slot(k):
        pltpu.sync_copy(slots.at[k, pl.ds(s * SLICE, SLICE)], red_t)

        @pl.loop(0, SLICE // L, unroll=8)
        def _add(i):
            red_a[pl.ds(i * L, L)] = red_a[pl.ds(i * L, L)] \
                + red_t[pl.ds(i * L, L)]

    pltpu.sync_copy(red_a, out_h.at[pl.ds(c * ACC_N + s * SLICE, SLICE)])


def kernel(img):
    out = _ct_project_sc(img, jnp.asarray(_PAR), jnp.asarray(_XTAB),
                         jnp.asarray(_YTAB))
    return out.reshape(NTHETA, Nu)
